# Initial kernel scaffold; baseline (speedup 1.0000x reference)
import jax
import jax.numpy as jnp
from jax.experimental import pallas as pl


def kernel(ego_emb, side_emb, Wq, bq, Wk, bk, Wv, bv):
    raise NotImplementedError("write your pallas kernel here")


# speedup vs baseline: 1.9891x; 1.9891x over previous
import functools

import jax
import jax.numpy as jnp
import numpy as np
from jax.experimental import pallas as pl
from jax.experimental.pallas import tpu as pltpu

_D = 256
_K = 16
_ROW_BLK = 256
_NEG = np.float32(-1e30)


def _proj_body(side_ref, wk_ref, bk_ref, wv_ref, bv_ref, k_ref, v_ref):
    s = side_ref[...]
    k_ref[...] = jnp.dot(s, wk_ref[...], preferred_element_type=jnp.float32) + bk_ref[...]
    v_ref[...] = jnp.dot(s, wv_ref[...], preferred_element_type=jnp.float32) + bv_ref[...]


def _attend_body(n_valid, n_pad, ego_ref, wq_ref, bq_ref, kmat_ref, vmat_ref, out_ref):
    rows = ego_ref.shape[0]
    q = jnp.dot(ego_ref[...], wq_ref[...], preferred_element_type=jnp.float32) + bq_ref[...]
    scale = jnp.float32(1.0 / np.sqrt(_D))
    s = jax.lax.dot_general(
        q, kmat_ref[...], (((1,), (1,)), ((), ())),
        preferred_element_type=jnp.float32) * scale
    col = jax.lax.broadcasted_iota(jnp.int32, (rows, n_pad), 1)
    s = jnp.where(col < n_valid, s, _NEG)

    m1 = jnp.max(s, axis=1, keepdims=True)

    def body(_, carry):
        s_c, w_c, z_c = carry
        m = jnp.max(s_c, axis=1, keepdims=True)
        hit = s_c == m
        idx = jnp.min(jnp.where(hit, col, n_pad), axis=1, keepdims=True)
        sel = col == idx
        e = jnp.exp(m - m1)
        w_c = jnp.where(sel, e, w_c)
        z_c = z_c + e
        s_c = jnp.where(sel, _NEG, s_c)
        return s_c, w_c, z_c

    w0 = jnp.zeros((rows, n_pad), jnp.float32)
    z0 = jnp.zeros((rows, 1), jnp.float32)
    _, w, z = jax.lax.fori_loop(0, _K, body, (s, w0, z0))
    agg = jnp.dot(w, vmat_ref[...], preferred_element_type=jnp.float32)
    out_ref[...] = agg / z


def _run(ego_emb, side_emb, Wq, bq, Wk, bk, Wv, bv, interpret=False):
    n, d = ego_emb.shape
    n_pad = ((n + _ROW_BLK - 1) // _ROW_BLK) * _ROW_BLK
    grid = n_pad // _ROW_BLK
    ego_p = jnp.pad(ego_emb, ((0, n_pad - n), (0, 0)))
    side_p = jnp.pad(side_emb, ((0, n_pad - n), (0, 0)))

    kmat, vmat = pl.pallas_call(
        _proj_body,
        grid=(grid,),
        in_specs=[
            pl.BlockSpec((_ROW_BLK, d), lambda i: (i, 0)),
            pl.BlockSpec((d, d), lambda i: (0, 0)),
            pl.BlockSpec((1, d), lambda i: (0, 0)),
            pl.BlockSpec((d, d), lambda i: (0, 0)),
            pl.BlockSpec((1, d), lambda i: (0, 0)),
        ],
        out_specs=[
            pl.BlockSpec((_ROW_BLK, d), lambda i: (i, 0)),
            pl.BlockSpec((_ROW_BLK, d), lambda i: (i, 0)),
        ],
        out_shape=[
            jax.ShapeDtypeStruct((n_pad, d), jnp.float32),
            jax.ShapeDtypeStruct((n_pad, d), jnp.float32),
        ],
        interpret=interpret,
    )(side_p, Wk, bk[None, :], Wv, bv[None, :])

    out = pl.pallas_call(
        functools.partial(_attend_body, n, n_pad),
        grid=(grid,),
        in_specs=[
            pl.BlockSpec((_ROW_BLK, d), lambda i: (i, 0)),
            pl.BlockSpec((d, d), lambda i: (0, 0)),
            pl.BlockSpec((1, d), lambda i: (0, 0)),
            pl.BlockSpec((n_pad, d), lambda i: (0, 0)),
            pl.BlockSpec((n_pad, d), lambda i: (0, 0)),
        ],
        out_specs=pl.BlockSpec((_ROW_BLK, d), lambda i: (i, 0)),
        out_shape=jax.ShapeDtypeStruct((n_pad, d), jnp.float32),
        interpret=interpret,
    )(ego_p, Wq, bq[None, :], kmat, vmat)

    return out[:n]


def kernel(ego_emb, side_emb, Wq, bq, Wk, bk, Wv, bv):
    return _run(ego_emb, side_emb, Wq, bq, Wk, bk, Wv, bv)


# speedup vs baseline: 4.2669x; 2.1451x over previous
import functools

import jax
import jax.numpy as jnp
import numpy as np
from jax import lax
from jax.experimental import pallas as pl
from jax.experimental.pallas import tpu as pltpu
from jax.experimental.pallas import tpu_sc as plsc

_D = 256
_K = 16
_ROW_BLK = 256
_TILE = 128
_NEG = np.float32(-1e30)


def _proj_body(side_ref, wk_ref, bk_ref, wv_ref, bv_ref, k_ref, v_ref):
    s = side_ref[...]
    k_ref[...] = jnp.dot(s, wk_ref[...], preferred_element_type=jnp.float32) + bk_ref[...]
    v_ref[...] = jnp.dot(s, wv_ref[...], preferred_element_type=jnp.float32) + bv_ref[...]


def _score_body(n_valid, n_pad, ego_ref, wq_ref, bq_ref, kmat_ref,
                s_ref, t_ref, tau_ref):
    rows = ego_ref.shape[0]
    ntiles = n_pad // _TILE
    q = jnp.dot(ego_ref[...], wq_ref[...], preferred_element_type=jnp.float32) + bq_ref[...]
    scale = np.float32(1.0 / np.sqrt(_D))
    s = lax.dot_general(
        q, kmat_ref[...], (((1,), (1,)), ((), ())),
        preferred_element_type=jnp.float32) * scale
    col = lax.broadcasted_iota(jnp.int32, (rows, n_pad), 1)
    s = jnp.where(col < n_valid, s, _NEG)
    s_ref[...] = s

    m3 = jnp.max(s.reshape(rows, ntiles, _TILE), axis=2)
    tcol = lax.broadcasted_iota(jnp.int32, (rows, ntiles), 1)
    tl = []
    m = None
    for _ in range(_K):
        m = jnp.max(m3, axis=1, keepdims=True)
        hit = m3 == m
        tidx = jnp.min(jnp.where(hit, tcol, ntiles), axis=1, keepdims=True)
        tl.append(tidx)
        m3 = jnp.where(tcol == tidx, _NEG, m3)
    t_ref[...] = jnp.concatenate(tl, axis=1)
    tau_ref[...] = m


def _sc_attend_body(n_pad, rows_per_w, num_cores,
                    s_hbm, t_hbm, tau_hbm, v_hbm, out_hbm,
                    t_v, tau_v, tiles_v, idx_v, vrows_v, out_v, bc_v, sem):
    ntiles = n_pad // _TILE
    wid = lax.axis_index("s") * num_cores + lax.axis_index("c")
    base = wid * rows_per_w
    pltpu.sync_copy(t_hbm.at[pl.ds(base * _K, rows_per_w * _K)], t_v)
    pltpu.sync_copy(tau_hbm.at[pl.ds(base, rows_per_w)], tau_v)
    iota16 = lax.iota(jnp.int32, 16)

    def row_body(r_loc, carry):
        r_glob = base + r_loc
        t_row = plsc.load_gather(t_v, [r_loc * _K + iota16])
        idx_v[...] = t_row + r_glob * ntiles
        pltpu.async_copy(s_hbm.at[idx_v], tiles_v, sem).wait()
        tau_s = plsc.load_gather(tau_v, [jnp.full((16,), r_loc, jnp.int32)])

        rv = jnp.full((16,), _NEG, jnp.float32)
        ri = jnp.zeros((16,), jnp.int32)
        for j in range(_K):
            tb = plsc.load_gather(t_v, [jnp.full((16,), r_loc * _K + j, jnp.int32)])
            for c in range(_TILE // 16):
                val = tiles_v[j, pl.ds(c * 16, 16)]
                colid = tb * _TILE + (c * 16) + iota16
                pred = jnp.any(val >= tau_s)

                def _merge(op):
                    rv0, ri0, v0, c0 = op
                    sv, si = plsc.sort_key_val(v0, c0, descending=True)
                    rrev = lax.rev(rv0, (0,))
                    irev = lax.rev(ri0, (0,))
                    mk = sv >= rrev
                    nv = jnp.where(mk, sv, rrev)
                    ni = jnp.where(mk, si, irev)
                    nv, ni = plsc.sort_key_val(nv, ni, descending=True)
                    return nv, ni, v0, c0

                rv, ri, _, _ = lax.cond(pred, _merge, lambda op: op,
                                        (rv, ri, val, colid))

        m1 = jnp.full((16,), jnp.max(rv), jnp.float32)
        w = jnp.exp(rv - m1)
        z = jnp.full((16,), jnp.sum(w), jnp.float32)
        wn = w / z

        idx_v[...] = ri
        pltpu.async_copy(v_hbm.at[idx_v], vrows_v, sem).wait()

        bc_v[pl.ds(16, 16)] = wn
        accs = [jnp.zeros((16,), jnp.float32) for _ in range(_D // 16)]
        for j in range(_K):
            wb = plsc.load_gather(bc_v, [jnp.full((16,), 16 + j, jnp.int32)])
            for dch in range(_D // 16):
                accs[dch] = accs[dch] + wb * vrows_v[j, pl.ds(dch * 16, 16)]
        for dch in range(_D // 16):
            out_v[pl.ds(dch * 16, 16)] = accs[dch]
        pltpu.sync_copy(out_v, out_hbm.at[r_glob])
        return carry

    lax.fori_loop(0, rows_per_w, row_body, 0)


def _run_hybrid(ego_emb, side_emb, Wq, bq, Wk, bk, Wv, bv, interpret=False):
    n, d = ego_emb.shape
    n_pad = ((n + _ROW_BLK - 1) // _ROW_BLK) * _ROW_BLK
    grid = n_pad // _ROW_BLK
    ntiles = n_pad // _TILE
    ego_p = jnp.pad(ego_emb, ((0, n_pad - n), (0, 0)))
    side_p = jnp.pad(side_emb, ((0, n_pad - n), (0, 0)))

    kmat, vmat = pl.pallas_call(
        _proj_body,
        grid=(grid,),
        in_specs=[
            pl.BlockSpec((_ROW_BLK, d), lambda i: (i, 0)),
            pl.BlockSpec((d, d), lambda i: (0, 0)),
            pl.BlockSpec((1, d), lambda i: (0, 0)),
            pl.BlockSpec((d, d), lambda i: (0, 0)),
            pl.BlockSpec((1, d), lambda i: (0, 0)),
        ],
        out_specs=[
            pl.BlockSpec((_ROW_BLK, d), lambda i: (i, 0)),
            pl.BlockSpec((_ROW_BLK, d), lambda i: (i, 0)),
        ],
        out_shape=[
            jax.ShapeDtypeStruct((n_pad, d), jnp.float32),
            jax.ShapeDtypeStruct((n_pad, d), jnp.float32),
        ],
        interpret=interpret,
    )(side_p, Wk, bk[None, :], Wv, bv[None, :])

    smat, tmat, tau = pl.pallas_call(
        functools.partial(_score_body, n, n_pad),
        grid=(grid,),
        in_specs=[
            pl.BlockSpec((_ROW_BLK, d), lambda i: (i, 0)),
            pl.BlockSpec((d, d), lambda i: (0, 0)),
            pl.BlockSpec((1, d), lambda i: (0, 0)),
            pl.BlockSpec((n_pad, d), lambda i: (0, 0)),
        ],
        out_specs=[
            pl.BlockSpec((_ROW_BLK, n_pad), lambda i: (i, 0)),
            pl.BlockSpec((_ROW_BLK, _K), lambda i: (i, 0)),
            pl.BlockSpec((_ROW_BLK, 1), lambda i: (i, 0)),
        ],
        out_shape=[
            jax.ShapeDtypeStruct((n_pad, n_pad), jnp.float32),
            jax.ShapeDtypeStruct((n_pad, _K), jnp.int32),
            jax.ShapeDtypeStruct((n_pad, 1), jnp.float32),
        ],
        interpret=interpret,
    )(ego_p, Wq, bq[None, :], kmat)

    try:
        info = plsc.get_sparse_core_info()
        num_cores, num_subcores = info.num_cores, info.num_subcores
    except Exception:
        num_cores, num_subcores = 2, 16
    num_workers = num_cores * num_subcores
    rows_per_w = n_pad // num_workers
    mesh = plsc.VectorSubcoreMesh(core_axis_name="c", subcore_axis_name="s",
                                  num_cores=num_cores, num_subcores=num_subcores)
    sc_fn = pl.kernel(
        functools.partial(_sc_attend_body, n_pad, rows_per_w, num_cores),
        mesh=mesh,
        compiler_params=pltpu.CompilerParams(needs_layout_passes=False),
        interpret=interpret,
        out_type=jax.ShapeDtypeStruct((n_pad, d), jnp.float32),
        scratch_types=[
            pltpu.VMEM((rows_per_w * _K,), jnp.int32),
            pltpu.VMEM((rows_per_w,), jnp.float32),
            pltpu.VMEM((_K, _TILE), jnp.float32),
            pltpu.VMEM((16,), jnp.int32),
            pltpu.VMEM((_K, d), jnp.float32),
            pltpu.VMEM((d,), jnp.float32),
            pltpu.VMEM((32,), jnp.float32),
            pltpu.SemaphoreType.DMA,
        ],
    )
    out = sc_fn(smat.reshape(n_pad * ntiles, _TILE),
                tmat.reshape(n_pad * _K),
                tau.reshape(n_pad),
                vmat)
    return out[:n]


def kernel(ego_emb, side_emb, Wq, bq, Wk, bk, Wv, bv):
    return _run_hybrid(ego_emb, side_emb, Wq, bq, Wk, bk, Wv, bv)
